# baseline (speedup 1.0000x reference)
import functools

import jax
import jax.numpy as jnp
from jax import lax
from jax.experimental import pallas as pl
from jax.experimental.pallas import tpu as pltpu
from jax.experimental.pallas import tpu_sc as plsc

_CLOSE_IDX = 1


def _phys_view_ochlv(ochlv, l, f, b, t):
    z = ochlv.transpose(2, 3, 1, 0)
    z = z.reshape(l, f, t // 128, 128, b)
    z = z.transpose(0, 1, 2, 4, 3)
    return z.reshape(l * f * b * t)


def _unphys_ochlv(y1d, l, f, b, t):
    z = y1d.reshape(l, f, t // 128, b, 128)
    z = z.transpose(3, 2, 4, 0, 1)
    return z.reshape(b, t, l, f)


def _phys_view_elem(e, b, t, d):
    z = e.transpose(0, 2, 1)
    z = z.reshape(b, d // 8, 8, t // 128, 128)
    z = z.transpose(0, 1, 3, 2, 4)
    return z.reshape(b * d * t)


def _unphys_elem(o1d, b, t, d):
    z = o1d.reshape(b, d // 8, t // 128, 8, 128)
    z = z.transpose(0, 2, 4, 1, 3)
    return z.reshape(b, t, d)


def _perf_body(x_ref, out_ref, minl_ref, startv_ref, *, lb, nsteps):
    step = pl.program_id(0)
    x = x_ref[:, 0]
    lio = lax.broadcasted_iota(jnp.int32, (lb, 1, 1, 1), 0) + step * lb
    big = jnp.int32(10 * lb * nsteps)
    l_or_big = jnp.where(x != 0.0, lio, big)
    minl_blk = jnp.min(l_or_big, axis=0)
    startv_blk = jnp.sum(jnp.where(l_or_big == minl_blk[None], x, 0.0), axis=0)

    @pl.when(step == 0)
    def _init():
        minl_ref[...] = minl_blk
        startv_ref[...] = startv_blk

    @pl.when(step > 0)
    def _merge():
        upd = minl_blk < minl_ref[...]
        minl_ref[...] = jnp.where(upd, minl_blk, minl_ref[...])
        startv_ref[...] = jnp.where(upd, startv_blk, startv_ref[...])

    @pl.when(step == nsteps - 1)
    def _finish():
        last = x_ref[lb - 1, 0]
        start = startv_ref[...]
        safe = jnp.where(start != 0.0, start, 1.0)
        out_ref[...] = jnp.where(start != 0.0, (last - start) / safe, 0.0)


def _compute_perf(x5):
    l, f, nt, b, tm = x5.shape
    lb = 8
    nsteps = l // lb
    return pl.pallas_call(
        functools.partial(_perf_body, lb=lb, nsteps=nsteps),
        grid=(nsteps,),
        in_specs=[pl.BlockSpec((lb, 1, nt, b, tm),
                               lambda i: (i, _CLOSE_IDX, 0, 0, 0))],
        out_specs=pl.BlockSpec((nt, b, tm), lambda i: (0, 0, 0)),
        out_shape=jax.ShapeDtypeStruct((nt, b, tm), jnp.float32),
        scratch_shapes=[
            pltpu.VMEM((nt, b, tm), jnp.int32),
            pltpu.VMEM((nt, b, tm), jnp.float32),
        ],
    )(x5)


def _sort_body(p_ref, orders_ref, pio_ref, pie_ref, rank_ref, *, nt, b, tm):
    p = p_ref[...]
    pj = p[:, :, None, :]
    jio = (lax.broadcasted_iota(jnp.int32, (nt, 1, 1, tm), 0) * tm
           + lax.broadcasted_iota(jnp.int32, (nt, 1, 1, tm), 3))
    for ti in range(nt):
        pi = p[ti][None, :, :, None]
        iio = (lax.broadcasted_iota(jnp.int32, (1, 1, tm, 1), 2) + ti * tm)
        cmp = (pj > pi) | ((pj == pi) & (jio < iio))
        c32 = cmp.astype(jnp.int32)
        rank_ref[ti] = jnp.sum(jnp.sum(c32, axis=3), axis=0)
    r = rank_ref[...][:, :, None, :]
    bio = lax.broadcasted_iota(jnp.int32, (b, 1), 0)
    for tk in range(nt):
        kio = (lax.broadcasted_iota(jnp.int32, (1, 1, tm, 1), 2) + tk * tm)
        hit = (r == kio)
        o = jnp.sum(jnp.sum(jnp.where(hit, jio, 0), axis=3), axis=0)
        orders_ref[:, tk * tm:(tk + 1) * tm] = o
        hi = o >> 7
        lo = o & 127
        pio_ref[tk] = (hi << 9) + (bio << 7) + lo
        pie_ref[tk] = (hi << 10) + lo


def _compute_orders(perf_phys):
    nt, b, tm = perf_phys.shape
    t = nt * tm
    return pl.pallas_call(
        functools.partial(_sort_body, nt=nt, b=b, tm=tm),
        in_specs=[pl.BlockSpec((nt, b, tm), lambda: (0, 0, 0))],
        out_specs=[
            pl.BlockSpec((b, t), lambda: (0, 0)),
            pl.BlockSpec((nt, b, tm), lambda: (0, 0, 0)),
            pl.BlockSpec((nt, b, tm), lambda: (0, 0, 0)),
        ],
        out_shape=[
            jax.ShapeDtypeStruct((b, t), jnp.int32),
            jax.ShapeDtypeStruct((nt, b, tm), jnp.int32),
            jax.ShapeDtypeStruct((nt, b, tm), jnp.int32),
        ],
        scratch_shapes=[pltpu.VMEM((nt, b, tm), jnp.int32)],
    )(perf_phys)


def _make_sc_gather(b, t, n_planes, d):
    info = plsc.get_sparse_core_info()
    nc, ns = info.num_cores, info.num_subcores
    nw = nc * ns
    ppw = n_planes // nw
    pw = b * t
    ew = 8 * t
    n_oc = n_planes * pw
    n_e = b * d * t

    mesh = plsc.VectorSubcoreMesh(core_axis_name="c", subcore_axis_name="s")

    @functools.partial(
        pl.kernel,
        mesh=mesh,
        compiler_params=pltpu.CompilerParams(needs_layout_passes=False),
        out_type=[
            jax.ShapeDtypeStruct((n_oc,), jnp.float32),
            jax.ShapeDtypeStruct((n_e,), jnp.float32),
            jax.ShapeDtypeStruct((n_e,), jnp.float32),
        ],
        scratch_types=[
            pltpu.VMEM((b * t,), jnp.int32),
            pltpu.VMEM((b * t,), jnp.int32),
            pltpu.VMEM((4 * pw,), jnp.float32),
            pltpu.VMEM((4 * pw,), jnp.float32),
            pltpu.VMEM((ew,), jnp.float32),
            pltpu.VMEM((ew,), jnp.float32),
            pltpu.SemaphoreType.DMA,
            pltpu.SemaphoreType.DMA,
            pltpu.SemaphoreType.DMA,
            pltpu.SemaphoreType.DMA,
            pltpu.SemaphoreType.DMA,
            pltpu.SemaphoreType.DMA,
            pltpu.SemaphoreType.DMA,
            pltpu.SemaphoreType.DMA,
        ],
    )
    def sc_gather(x_hbm, e0_hbm, e1_hbm, pio_hbm, pie_hbm,
                  y_hbm, o0_hbm, o1_hbm,
                  piov, piev, pin, pout, ein, eout,
                  sin0, sin1, sin2, sin3, sout0, sout1, sout2, sout3):
        wid = lax.axis_index("s") * nc + lax.axis_index("c")
        pltpu.sync_copy(pio_hbm, piov)
        pltpu.sync_copy(pie_hbm, piev)

        g0 = wid * ppw
        sins = (sin0, sin1, sin2, sin3)
        souts = (sout0, sout1, sout2, sout3)

        def permute_plane(src_ref, dst_ref):
            def jbody(jt, cc):
                jbase = jt << 9
                for bb in range(b):
                    for jm in range(t // (16 * 16)):
                        ofs = jbase + (bb << 7) + (jm << 4)
                        pv = piov[pl.ds(ofs, 16)]
                        dst_ref[pl.ds(ofs, 16)] = plsc.load_gather(
                            src_ref, [pv])
                return cc

            lax.fori_loop(0, t // 128, jbody, 0)

        nbuf = 4
        for q in range(nbuf - 1):
            pltpu.async_copy(
                x_hbm.at[pl.ds((g0 + q) * pw, pw)],
                pin.at[pl.ds(q * pw, pw)], sins[q])

        def plane_body(i, carry):
            for par in range(nbuf):
                p = g0 + nbuf * i + par
                nxt = jnp.minimum(p + (nbuf - 1), jnp.int32(n_planes - 1))
                nbi = (par + nbuf - 1) % nbuf
                pltpu.make_async_copy(
                    x_hbm.at[pl.ds(p * pw, pw)],
                    pin.at[pl.ds(par * pw, pw)], sins[par]).wait()
                pltpu.async_copy(
                    x_hbm.at[pl.ds(nxt * pw, pw)],
                    pin.at[pl.ds(nbi * pw, pw)], sins[nbi])

                @pl.when(i > 0)
                def _drain():
                    pltpu.make_async_copy(
                        pout.at[pl.ds(par * pw, pw)],
                        y_hbm.at[pl.ds(p * pw, pw)], souts[par]).wait()

                permute_plane(pin.at[pl.ds(par * pw, pw)],
                              pout.at[pl.ds(par * pw, pw)])
                pltpu.async_copy(
                    pout.at[pl.ds(par * pw, pw)],
                    y_hbm.at[pl.ds(p * pw, pw)], souts[par])
            return carry

        lax.fori_loop(0, ppw // nbuf, plane_body, 0)
        for q in range(nbuf):
            pltpu.make_async_copy(
                pout.at[pl.ds(q * pw, pw)],
                y_hbm.at[pl.ds(g0 * pw, pw)], souts[q]).wait()
        for q in range(nbuf - 1):
            pltpu.make_async_copy(
                x_hbm.at[pl.ds(g0 * pw, pw)],
                pin.at[pl.ds(q * pw, pw)], sins[q]).wait()

        ebase = wid * ew
        bq = wid // 8

        def permute_egroup(e_hbm, o_hbm, osem):
            pltpu.sync_copy(e_hbm.at[pl.ds(ebase, ew)], ein)

            def ejbody(jt, cc):
                sbase = (jt << 9) + (bq << 7)
                dbase = jt << 10
                for jm in range(t // (16 * 16)):
                    pv0 = piev[pl.ds(sbase + (jm << 4), 16)]
                    for k in range(8):
                        v = plsc.load_gather(ein, [pv0 + k * 128])
                        eout[pl.ds(dbase + (k << 7) + (jm << 4), 16)] = v
                return cc

            lax.fori_loop(0, t // 128, ejbody, 0)
            pltpu.async_copy(eout, o_hbm.at[pl.ds(ebase, ew)], osem)

        permute_egroup(e0_hbm, o0_hbm, sout0)
        pltpu.make_async_copy(eout, o0_hbm.at[pl.ds(ebase, ew)], sout0).wait()
        permute_egroup(e1_hbm, o1_hbm, sout1)
        pltpu.make_async_copy(eout, o1_hbm.at[pl.ds(ebase, ew)], sout1).wait()

    return sc_gather


def kernel(elem0, elem1, ochlv):
    b, t, l, f = ochlv.shape
    d = elem0.shape[-1]

    x1d = _phys_view_ochlv(ochlv, l, f, b, t)
    e0_1d = _phys_view_elem(elem0, b, t, d)
    e1_1d = _phys_view_elem(elem1, b, t, d)

    x5 = x1d.reshape(l, f, t // 128, b, 128)
    perf_phys = _compute_perf(x5)
    orders, pio, pie = _compute_orders(perf_phys)

    sc_gather = _make_sc_gather(b, t, l * f, d)
    y1d, o0_1d, o1_1d = sc_gather(
        x1d, e0_1d, e1_1d, pio.reshape(b * t), pie.reshape(b * t))

    o0 = _unphys_elem(o0_1d, b, t, d)
    o1 = _unphys_elem(o1_1d, b, t, d)
    o2 = _unphys_ochlv(y1d, l, f, b, t)
    return (o0, o1, o2, orders)


# speedup vs baseline: 1.0528x; 1.0528x over previous
import functools

import jax
import jax.numpy as jnp
from jax import lax
from jax.experimental import pallas as pl
from jax.experimental.pallas import tpu as pltpu
from jax.experimental.pallas import tpu_sc as plsc

_CLOSE_IDX = 1


def _phys_view_ochlv(ochlv, l, f, b, t):
    z = ochlv.transpose(2, 3, 1, 0)
    z = z.reshape(l, f, t // 128, 128, b)
    z = z.transpose(0, 1, 2, 4, 3)
    return z.reshape(l * f * b * t)


def _unphys_ochlv(y1d, l, f, b, t):
    z = y1d.reshape(l, f, t // 128, b, 128)
    z = z.transpose(3, 2, 4, 0, 1)
    return z.reshape(b, t, l, f)


def _phys_view_elem(e, b, t, d):
    z = e.transpose(0, 2, 1)
    z = z.reshape(b, d // 8, 8, t // 128, 128)
    z = z.transpose(0, 1, 3, 2, 4)
    return z.reshape(b * d * t)


def _unphys_elem(o1d, b, t, d):
    z = o1d.reshape(b, d // 8, t // 128, 8, 128)
    z = z.transpose(0, 2, 4, 1, 3)
    return z.reshape(b, t, d)


def _perf_body(x_ref, out_ref, minl_ref, startv_ref, *, lb, nsteps):
    step = pl.program_id(0)
    x = x_ref[:, 0]
    lio = lax.broadcasted_iota(jnp.int32, (lb, 1, 1, 1), 0) + step * lb
    big = jnp.int32(10 * lb * nsteps)
    l_or_big = jnp.where(x != 0.0, lio, big)
    minl_blk = jnp.min(l_or_big, axis=0)
    startv_blk = jnp.sum(jnp.where(l_or_big == minl_blk[None], x, 0.0), axis=0)

    @pl.when(step == 0)
    def _init():
        minl_ref[...] = minl_blk
        startv_ref[...] = startv_blk

    @pl.when(step > 0)
    def _merge():
        upd = minl_blk < minl_ref[...]
        minl_ref[...] = jnp.where(upd, minl_blk, minl_ref[...])
        startv_ref[...] = jnp.where(upd, startv_blk, startv_ref[...])

    @pl.when(step == nsteps - 1)
    def _finish():
        last = x_ref[lb - 1, 0]
        start = startv_ref[...]
        safe = jnp.where(start != 0.0, start, 1.0)
        out_ref[...] = jnp.where(start != 0.0, (last - start) / safe, 0.0)


def _compute_perf(x5):
    l, f, nt, b, tm = x5.shape
    lb = 32
    nsteps = l // lb
    return pl.pallas_call(
        functools.partial(_perf_body, lb=lb, nsteps=nsteps),
        grid=(nsteps,),
        in_specs=[pl.BlockSpec((lb, 1, nt, b, tm),
                               lambda i: (i, _CLOSE_IDX, 0, 0, 0))],
        out_specs=pl.BlockSpec((nt, b, tm), lambda i: (0, 0, 0)),
        out_shape=jax.ShapeDtypeStruct((nt, b, tm), jnp.float32),
        scratch_shapes=[
            pltpu.VMEM((nt, b, tm), jnp.int32),
            pltpu.VMEM((nt, b, tm), jnp.float32),
        ],
    )(x5)


def _sort_body(p_ref, orders_ref, pio_ref, pie_ref, rank_ref, *, nt, b, tm):
    p = p_ref[...]
    pj = p[:, :, None, :]
    jio = (lax.broadcasted_iota(jnp.int32, (nt, 1, 1, tm), 0) * tm
           + lax.broadcasted_iota(jnp.int32, (nt, 1, 1, tm), 3))
    for ti in range(nt):
        pi = p[ti][None, :, :, None]
        iio = (lax.broadcasted_iota(jnp.int32, (1, 1, tm, 1), 2) + ti * tm)
        cmp = (pj > pi) | ((pj == pi) & (jio < iio))
        c32 = cmp.astype(jnp.int32)
        rank_ref[ti] = jnp.sum(jnp.sum(c32, axis=3), axis=0)
    r = rank_ref[...][:, :, None, :]
    bio = lax.broadcasted_iota(jnp.int32, (b, 1), 0)
    for tk in range(nt):
        kio = (lax.broadcasted_iota(jnp.int32, (1, 1, tm, 1), 2) + tk * tm)
        hit = (r == kio)
        o = jnp.sum(jnp.sum(jnp.where(hit, jio, 0), axis=3), axis=0)
        orders_ref[:, tk * tm:(tk + 1) * tm] = o
        hi = o >> 7
        lo = o & 127
        pio_ref[tk] = (hi << 9) + (bio << 7) + lo
        pie_ref[tk] = (hi << 10) + lo


def _compute_orders(perf_phys):
    nt, b, tm = perf_phys.shape
    t = nt * tm
    return pl.pallas_call(
        functools.partial(_sort_body, nt=nt, b=b, tm=tm),
        in_specs=[pl.BlockSpec((nt, b, tm), lambda: (0, 0, 0))],
        out_specs=[
            pl.BlockSpec((b, t), lambda: (0, 0)),
            pl.BlockSpec((nt, b, tm), lambda: (0, 0, 0)),
            pl.BlockSpec((nt, b, tm), lambda: (0, 0, 0)),
        ],
        out_shape=[
            jax.ShapeDtypeStruct((b, t), jnp.int32),
            jax.ShapeDtypeStruct((nt, b, tm), jnp.int32),
            jax.ShapeDtypeStruct((nt, b, tm), jnp.int32),
        ],
        scratch_shapes=[pltpu.VMEM((nt, b, tm), jnp.int32)],
    )(perf_phys)


def _make_sc_gather(b, t, n_planes, d):
    info = plsc.get_sparse_core_info()
    nc, ns = info.num_cores, info.num_subcores
    nw = nc * ns
    ppw = n_planes // nw
    pw = b * t
    ew = 8 * t
    n_oc = n_planes * pw
    n_e = b * d * t

    mesh = plsc.VectorSubcoreMesh(core_axis_name="c", subcore_axis_name="s")

    @functools.partial(
        pl.kernel,
        mesh=mesh,
        compiler_params=pltpu.CompilerParams(needs_layout_passes=False),
        out_type=[
            jax.ShapeDtypeStruct((n_oc,), jnp.float32),
            jax.ShapeDtypeStruct((n_e,), jnp.float32),
            jax.ShapeDtypeStruct((n_e,), jnp.float32),
        ],
        scratch_types=[
            pltpu.VMEM((b * t,), jnp.int32),
            pltpu.VMEM((b * t,), jnp.int32),
            pltpu.VMEM((4 * pw,), jnp.float32),
            pltpu.VMEM((4 * pw,), jnp.float32),
            pltpu.VMEM((ew,), jnp.float32),
            pltpu.VMEM((ew,), jnp.float32),
            pltpu.SemaphoreType.DMA,
            pltpu.SemaphoreType.DMA,
            pltpu.SemaphoreType.DMA,
            pltpu.SemaphoreType.DMA,
            pltpu.SemaphoreType.DMA,
            pltpu.SemaphoreType.DMA,
            pltpu.SemaphoreType.DMA,
            pltpu.SemaphoreType.DMA,
        ],
    )
    def sc_gather(x_hbm, e0_hbm, e1_hbm, pio_hbm, pie_hbm,
                  y_hbm, o0_hbm, o1_hbm,
                  piov, piev, pin, pout, ein, eout,
                  sin0, sin1, sin2, sin3, sout0, sout1, sout2, sout3):
        wid = lax.axis_index("s") * nc + lax.axis_index("c")
        pltpu.sync_copy(pio_hbm, piov)
        pltpu.sync_copy(pie_hbm, piev)

        g0 = wid * ppw
        sins = (sin0, sin1, sin2, sin3)
        souts = (sout0, sout1, sout2, sout3)

        def permute_plane2(src_ref, dst_ref):
            def jbody(jt, cc):
                jbase = jt << 9
                for bb in range(b):
                    for jm in range(t // (16 * 16)):
                        ofs = jbase + (bb << 7) + (jm << 4)
                        pv = piov[pl.ds(ofs, 16)]
                        dst_ref[pl.ds(ofs, 16)] = plsc.load_gather(
                            src_ref, [pv])
                        dst_ref[pl.ds(pw + ofs, 16)] = plsc.load_gather(
                            src_ref, [pv + pw])
                return cc

            lax.fori_loop(0, t // 128, jbody, 0)

        nslot = 2
        spw = 2 * pw
        npair = ppw // 2
        for q in range(nslot - 1):
            pltpu.async_copy(
                x_hbm.at[pl.ds((g0 + 2 * q) * pw, spw)],
                pin.at[pl.ds(q * spw, spw)], sins[q])

        def plane_body(i, carry):
            for par in range(nslot):
                pr = nslot * i + par
                p0 = g0 + 2 * pr
                nxt = g0 + 2 * jnp.minimum(pr + (nslot - 1),
                                           jnp.int32(npair - 1))
                nbi = (par + nslot - 1) % nslot
                pltpu.make_async_copy(
                    x_hbm.at[pl.ds(p0 * pw, spw)],
                    pin.at[pl.ds(par * spw, spw)], sins[par]).wait()
                pltpu.async_copy(
                    x_hbm.at[pl.ds(nxt * pw, spw)],
                    pin.at[pl.ds(nbi * spw, spw)], sins[nbi])

                @pl.when(i > 0)
                def _drain():
                    pltpu.make_async_copy(
                        pout.at[pl.ds(par * spw, spw)],
                        y_hbm.at[pl.ds(p0 * pw, spw)], souts[par]).wait()

                permute_plane2(pin.at[pl.ds(par * spw, spw)],
                               pout.at[pl.ds(par * spw, spw)])
                pltpu.async_copy(
                    pout.at[pl.ds(par * spw, spw)],
                    y_hbm.at[pl.ds(p0 * pw, spw)], souts[par])
            return carry

        lax.fori_loop(0, npair // nslot, plane_body, 0)
        for q in range(nslot):
            pltpu.make_async_copy(
                pout.at[pl.ds(q * spw, spw)],
                y_hbm.at[pl.ds(g0 * pw, spw)], souts[q]).wait()
        for q in range(nslot - 1):
            pltpu.make_async_copy(
                x_hbm.at[pl.ds(g0 * pw, spw)],
                pin.at[pl.ds(q * spw, spw)], sins[q]).wait()

        ebase = wid * ew
        bq = wid // 8

        def permute_egroup(e_hbm, o_hbm, osem):
            pltpu.sync_copy(e_hbm.at[pl.ds(ebase, ew)], ein)

            def ejbody(jt, cc):
                sbase = (jt << 9) + (bq << 7)
                dbase = jt << 10
                for jm in range(t // (16 * 16)):
                    pv0 = piev[pl.ds(sbase + (jm << 4), 16)]
                    for k in range(8):
                        v = plsc.load_gather(ein, [pv0 + k * 128])
                        eout[pl.ds(dbase + (k << 7) + (jm << 4), 16)] = v
                return cc

            lax.fori_loop(0, t // 128, ejbody, 0)
            pltpu.async_copy(eout, o_hbm.at[pl.ds(ebase, ew)], osem)

        permute_egroup(e0_hbm, o0_hbm, sout0)
        pltpu.make_async_copy(eout, o0_hbm.at[pl.ds(ebase, ew)], sout0).wait()
        permute_egroup(e1_hbm, o1_hbm, sout1)
        pltpu.make_async_copy(eout, o1_hbm.at[pl.ds(ebase, ew)], sout1).wait()

    return sc_gather


def kernel(elem0, elem1, ochlv):
    b, t, l, f = ochlv.shape
    d = elem0.shape[-1]

    x1d = _phys_view_ochlv(ochlv, l, f, b, t)
    e0_1d = _phys_view_elem(elem0, b, t, d)
    e1_1d = _phys_view_elem(elem1, b, t, d)

    x5 = x1d.reshape(l, f, t // 128, b, 128)
    perf_phys = _compute_perf(x5)
    orders, pio, pie = _compute_orders(perf_phys)

    sc_gather = _make_sc_gather(b, t, l * f, d)
    y1d, o0_1d, o1_1d = sc_gather(
        x1d, e0_1d, e1_1d, pio.reshape(b * t), pie.reshape(b * t))

    o0 = _unphys_elem(o0_1d, b, t, d)
    o1 = _unphys_elem(o1_1d, b, t, d)
    o2 = _unphys_ochlv(y1d, l, f, b, t)
    return (o0, o1, o2, orders)
